# baseline (speedup 1.0000x reference)
import jax
import jax.numpy as jnp
from jax import lax
from jax.experimental import pallas as pl
from jax.experimental.pallas import tpu as pltpu
from jax.experimental.pallas import tpu_sc as plsc

_N = 10000
_E = 320000
_HID = 128
_OUT = 10
_G = 64

_RB = 1000
_GRID = _N // _RB

_NC, _NS = 2, 16
_K = 125
_EPT = _E // (_NC * _NS)
_CHUNKS = _EPT // _K
_GP = 8
_NGRP = _CHUNKS // _GP
_NPAD = 10240
_RPT = _NPAD // _NS
_RCH = 80


def _tc1_body(x_ref, w_ref, xw_ref, t_ref):
    xw = jnp.dot(x_ref[...], w_ref[...], preferred_element_type=jnp.float32)
    xw_ref[...] = xw
    t_ref[...] = jnp.tanh(jnp.maximum(xw, 0.0))


def _tc1(x, w_self_t):
    return pl.pallas_call(
        _tc1_body,
        grid=(_GRID,),
        in_specs=[
            pl.BlockSpec((_RB, _HID), lambda i: (i, 0)),
            pl.BlockSpec((_HID, _HID), lambda i: (0, 0)),
        ],
        out_specs=[
            pl.BlockSpec((_RB, _HID), lambda i: (i, 0)),
            pl.BlockSpec((_RB, _HID), lambda i: (i, 0)),
        ],
        out_shape=[
            jax.ShapeDtypeStruct((_N, _HID), jnp.float32),
            jax.ShapeDtypeStruct((_N, _HID), jnp.float32),
        ],
    )(x, w_self_t)


def _sc_body(src_ref, dst_ref, t_ref, out_ref, sia, dia, sib, dib, gba, gbb,
             acc, semga, semgb, semsa, semsb, semi):
    cid = lax.axis_index("c")
    sid = lax.axis_index("s")

    def _zrow(r, c0):
        def _zcol(c, c1):
            gba[r, pl.ds(c * 16, 16)] = jnp.zeros((16,), jnp.float32)
            return c1
        return lax.fori_loop(0, _HID // 16, _zcol, c0)

    lax.fori_loop(0, _RCH, _zrow, 0)

    def _zslice(q, c0):
        pltpu.sync_copy(gba.at[pl.ds(0, _RCH)],
                        acc.at[pl.ds(sid * _RPT + q * _RCH, _RCH)])
        return c0

    lax.fori_loop(0, _RPT // _RCH, _zslice, 0)
    plsc.subcore_barrier()

    pltpu.sync_copy(src_ref.at[cid, sid, 0], sia)
    pltpu.sync_copy(dst_ref.at[cid, sid, 0], dia)
    pltpu.async_copy(t_ref.at[sia.at[0]], gba, semga)

    def _do_group(cs, cd, crossing, skip_pred):
        for r in range(_GP):
            if r % 2 == 0:
                buf, semg, sems = gba, semga, semsa
                obuf, osemg, osems = gbb, semgb, semsb
            else:
                buf, semg, sems = gbb, semgb, semsb
                obuf, osemg, osems = gba, semga, semsa
            pltpu.make_async_copy(t_ref.at[cs.at[r]], buf, semg).wait()
            pltpu.async_copy(buf, acc.at[cd.at[r]], sems, add=True)

            def _wait_prev(ob=obuf, os=osems, rr=r):
                pltpu.make_async_copy(ob, acc.at[cd.at[rr]], os).wait()

            if r == 0 and skip_pred is not None:
                @pl.when(jnp.logical_not(skip_pred))
                def _():
                    _wait_prev()
            else:
                _wait_prev()
            if r < _GP - 1:
                pltpu.async_copy(t_ref.at[cs.at[r + 1]], obuf, osemg)
            elif crossing is not None:
                crossing()

    def _cross(ns, nd):
        pltpu.make_async_copy(src_ref.at[cid, sid, 0], ns, semi).wait()
        pltpu.make_async_copy(dst_ref.at[cid, sid, 0], nd, semi).wait()
        pltpu.async_copy(t_ref.at[ns.at[0]], gba, semga)

    def _pairbody(gg, c0):
        g0 = 2 * gg
        pltpu.async_copy(src_ref.at[cid, sid, g0 + 1], sib, semi)
        pltpu.async_copy(dst_ref.at[cid, sid, g0 + 1], dib, semi)
        _do_group(sia, dia, lambda: _cross(sib, dib), gg == 0)

        @pl.when(gg < _NGRP // 2 - 1)
        def _():
            pltpu.async_copy(src_ref.at[cid, sid, g0 + 2], sia, semi)
            pltpu.async_copy(dst_ref.at[cid, sid, g0 + 2], dia, semi)

        def _cross_if_more():
            @pl.when(gg < _NGRP // 2 - 1)
            def _():
                _cross(sia, dia)

        _do_group(sib, dib, _cross_if_more, None)
        return c0

    lax.fori_loop(0, _NGRP // 2, _pairbody, 0)
    pltpu.make_async_copy(gbb, acc.at[dib.at[_GP - 1]], semsb).wait()
    plsc.subcore_barrier()

    def _cp(q, c0):
        r0 = sid * _RPT + q * _RCH
        pltpu.sync_copy(acc.at[pl.ds(r0, _RCH)], gba.at[pl.ds(0, _RCH)])
        pltpu.sync_copy(gba.at[pl.ds(0, _RCH)], out_ref.at[cid, pl.ds(r0, _RCH)])
        return c0

    lax.fori_loop(0, _RPT // _RCH, _cp, 0)


def _sc_scatter(src5, dst5, t):
    mesh = plsc.VectorSubcoreMesh(core_axis_name="c", subcore_axis_name="s")
    f = pl.kernel(
        _sc_body,
        mesh=mesh,
        out_type=jax.ShapeDtypeStruct((_NC, _NPAD, _HID), jnp.float32),
        scratch_types=[
            pltpu.VMEM((_GP, _K), jnp.int32),
            pltpu.VMEM((_GP, _K), jnp.int32),
            pltpu.VMEM((_GP, _K), jnp.int32),
            pltpu.VMEM((_GP, _K), jnp.int32),
            pltpu.VMEM((_K, _HID), jnp.float32),
            pltpu.VMEM((_K, _HID), jnp.float32),
            pltpu.VMEM_SHARED((_NPAD, _HID), jnp.float32),
            pltpu.SemaphoreType.DMA,
            pltpu.SemaphoreType.DMA,
            pltpu.SemaphoreType.DMA,
            pltpu.SemaphoreType.DMA,
            pltpu.SemaphoreType.DMA,
        ],
    )
    return f(src5, dst5, t)


def _tc2_body(xw_ref, agg_ref, wn_ref, b_ref, wp_ref, bp_ref, out_ref,
              pooled, counts):
    i = pl.program_id(0)

    @pl.when(i == 0)
    def _():
        pooled[...] = jnp.zeros_like(pooled)
        counts[...] = jnp.zeros_like(counts)

    agg = agg_ref[0] + agg_ref[1]
    o = jnp.maximum(
        xw_ref[...]
        + jnp.dot(jnp.tanh(agg), wn_ref[...], preferred_element_type=jnp.float32),
        0.0,
    )
    b = b_ref[0, 0, :]
    onehot = (lax.broadcasted_iota(jnp.int32, (_G, _RB), 0) == b[None, :])
    onehot = onehot.astype(jnp.float32)
    pooled[...] += jnp.dot(onehot, o, preferred_element_type=jnp.float32)
    counts[...] += jnp.sum(onehot, axis=1, keepdims=True)

    @pl.when(i == _GRID - 1)
    def _():
        pm = pooled[...] / jnp.maximum(counts[...], 1.0)
        logits = jnp.dot(pm, wp_ref[...], preferred_element_type=jnp.float32)
        logits = logits + bp_ref[...]
        m = jnp.max(logits, axis=-1, keepdims=True)
        lse = jnp.log(jnp.sum(jnp.exp(logits - m), axis=-1, keepdims=True)) + m
        out_ref[...] = logits - lse


def _tc2(xw, agg2, w_nbr_t, batch_r, w_pred_t, b_pred_r):
    return pl.pallas_call(
        _tc2_body,
        grid=(_GRID,),
        in_specs=[
            pl.BlockSpec((_RB, _HID), lambda i: (i, 0)),
            pl.BlockSpec((_NC, _RB, _HID), lambda i: (0, i, 0)),
            pl.BlockSpec((_HID, _HID), lambda i: (0, 0)),
            pl.BlockSpec((1, 1, _RB), lambda i: (i, 0, 0)),
            pl.BlockSpec((_HID, _OUT), lambda i: (0, 0)),
            pl.BlockSpec((1, _OUT), lambda i: (0, 0)),
        ],
        out_specs=pl.BlockSpec((_G, _OUT), lambda i: (0, 0)),
        out_shape=jax.ShapeDtypeStruct((_G, _OUT), jnp.float32),
        scratch_shapes=[
            pltpu.VMEM((_G, _HID), jnp.float32),
            pltpu.VMEM((_G, 1), jnp.float32),
        ],
    )(xw, agg2, w_nbr_t, batch_r, w_pred_t, b_pred_r)


def kernel(x, edge_index, batch, W_self, W_nbr, W_pred, b_pred):
    xw, t = _tc1(x, W_self.T)
    src5 = edge_index[0].reshape(_NC, _NS, _NGRP, _GP, _K)
    dst5 = edge_index[1].reshape(_NC, _NS, _NGRP, _GP, _K)
    agg2 = _sc_scatter(src5, dst5, t)
    batch_r = batch.reshape(_GRID, 1, _RB)
    return _tc2(xw, agg2, W_nbr.T, batch_r, W_pred.T, b_pred.reshape(1, _OUT))


# speedup vs baseline: 1.0583x; 1.0583x over previous
import jax
import jax.numpy as jnp
from jax import lax
from jax.experimental import pallas as pl
from jax.experimental.pallas import tpu as pltpu
from jax.experimental.pallas import tpu_sc as plsc

_N = 10000
_E = 320000
_HID = 128
_OUT = 10
_G = 64

_RB = 1000
_GRID = _N // _RB

_NC, _NS = 2, 16
_K = 125
_EPT = _E // (_NC * _NS)
_CHUNKS = _EPT // _K
_GP = 8
_NGRP = _CHUNKS // _GP
_NPAD = 10240
_RPT = _NPAD // _NS
_RCH = 80


def _tc1_body(x_ref, w_ref, xw_ref, t_ref):
    xw = jnp.dot(x_ref[...], w_ref[...], preferred_element_type=jnp.float32)
    xw_ref[...] = xw
    t_ref[...] = jnp.tanh(jnp.maximum(xw, 0.0))


def _tc1(x, w_self_t):
    return pl.pallas_call(
        _tc1_body,
        grid=(_GRID,),
        in_specs=[
            pl.BlockSpec((_RB, _HID), lambda i: (i, 0)),
            pl.BlockSpec((_HID, _HID), lambda i: (0, 0)),
        ],
        out_specs=[
            pl.BlockSpec((_RB, _HID), lambda i: (i, 0)),
            pl.BlockSpec((_RB, _HID), lambda i: (i, 0)),
        ],
        out_shape=[
            jax.ShapeDtypeStruct((_N, _HID), jnp.float32),
            jax.ShapeDtypeStruct((_N, _HID), jnp.float32),
        ],
    )(x, w_self_t)


def _sc_body(e_ref, t_ref, out_ref, sia, dia, sib, dib, gba, gbb,
             acc, semga, semgb, semsa, semsb, semi):
    cid = lax.axis_index("c")
    sid = lax.axis_index("s")
    src_ref = e_ref.at[0]
    dst_ref = e_ref.at[1]

    def _zrow(r, c0):
        def _zcol(c, c1):
            gba[r, pl.ds(c * 16, 16)] = jnp.zeros((16,), jnp.float32)
            return c1
        return lax.fori_loop(0, _HID // 16, _zcol, c0)

    lax.fori_loop(0, _RCH, _zrow, 0)

    def _zslice(q, c0):
        pltpu.sync_copy(gba.at[pl.ds(0, _RCH)],
                        acc.at[pl.ds(sid * _RPT + q * _RCH, _RCH)])
        return c0

    lax.fori_loop(0, _RPT // _RCH, _zslice, 0)
    plsc.subcore_barrier()

    pltpu.sync_copy(src_ref.at[cid, sid, 0], sia)
    pltpu.sync_copy(dst_ref.at[cid, sid, 0], dia)
    pltpu.async_copy(t_ref.at[sia.at[0]], gba, semga)

    def _do_group(cs, cd, crossing, skip_pred):
        for r in range(_GP):
            if r % 2 == 0:
                buf, semg, sems = gba, semga, semsa
                obuf, osemg, osems = gbb, semgb, semsb
            else:
                buf, semg, sems = gbb, semgb, semsb
                obuf, osemg, osems = gba, semga, semsa
            pltpu.make_async_copy(t_ref.at[cs.at[r]], buf, semg).wait()
            pltpu.async_copy(buf, acc.at[cd.at[r]], sems, add=True)

            def _wait_prev(ob=obuf, os=osems, rr=r):
                pltpu.make_async_copy(ob, acc.at[cd.at[rr]], os).wait()

            if r == 0 and skip_pred is not None:
                @pl.when(jnp.logical_not(skip_pred))
                def _():
                    _wait_prev()
            else:
                _wait_prev()
            if r < _GP - 1:
                pltpu.async_copy(t_ref.at[cs.at[r + 1]], obuf, osemg)
            elif crossing is not None:
                crossing()

    def _cross(ns, nd):
        pltpu.make_async_copy(src_ref.at[cid, sid, 0], ns, semi).wait()
        pltpu.make_async_copy(dst_ref.at[cid, sid, 0], nd, semi).wait()
        pltpu.async_copy(t_ref.at[ns.at[0]], gba, semga)

    def _pairbody(gg, c0):
        g0 = 2 * gg
        pltpu.async_copy(src_ref.at[cid, sid, g0 + 1], sib, semi)
        pltpu.async_copy(dst_ref.at[cid, sid, g0 + 1], dib, semi)
        _do_group(sia, dia, lambda: _cross(sib, dib), gg == 0)

        @pl.when(gg < _NGRP // 2 - 1)
        def _():
            pltpu.async_copy(src_ref.at[cid, sid, g0 + 2], sia, semi)
            pltpu.async_copy(dst_ref.at[cid, sid, g0 + 2], dia, semi)

        def _cross_if_more():
            @pl.when(gg < _NGRP // 2 - 1)
            def _():
                _cross(sia, dia)

        _do_group(sib, dib, _cross_if_more, None)
        return c0

    lax.fori_loop(0, _NGRP // 2, _pairbody, 0)
    pltpu.make_async_copy(gbb, acc.at[dib.at[_GP - 1]], semsb).wait()
    plsc.subcore_barrier()

    def _cp(q, c0):
        r0 = sid * _RPT + q * _RCH
        pltpu.sync_copy(acc.at[pl.ds(r0, _RCH)], gba.at[pl.ds(0, _RCH)])
        pltpu.sync_copy(gba.at[pl.ds(0, _RCH)], out_ref.at[cid, pl.ds(r0, _RCH)])
        return c0

    lax.fori_loop(0, _RPT // _RCH, _cp, 0)


def _sc_scatter(e6, t):
    mesh = plsc.VectorSubcoreMesh(core_axis_name="c", subcore_axis_name="s")
    f = pl.kernel(
        _sc_body,
        mesh=mesh,
        out_type=jax.ShapeDtypeStruct((_NC, _NPAD, _HID), jnp.float32),
        scratch_types=[
            pltpu.VMEM((_GP, _K), jnp.int32),
            pltpu.VMEM((_GP, _K), jnp.int32),
            pltpu.VMEM((_GP, _K), jnp.int32),
            pltpu.VMEM((_GP, _K), jnp.int32),
            pltpu.VMEM((_K, _HID), jnp.float32),
            pltpu.VMEM((_K, _HID), jnp.float32),
            pltpu.VMEM_SHARED((_NPAD, _HID), jnp.float32),
            pltpu.SemaphoreType.DMA,
            pltpu.SemaphoreType.DMA,
            pltpu.SemaphoreType.DMA,
            pltpu.SemaphoreType.DMA,
            pltpu.SemaphoreType.DMA,
        ],
    )
    return f(e6, t)


def _tc2_body(xw_ref, agg_ref, wn_ref, b_ref, wp_ref, bp_ref, out_ref,
              pooled, counts):
    i = pl.program_id(0)

    @pl.when(i == 0)
    def _():
        pooled[...] = jnp.zeros_like(pooled)
        counts[...] = jnp.zeros_like(counts)

    agg = agg_ref[0] + agg_ref[1]
    o = jnp.maximum(
        xw_ref[...]
        + jnp.dot(jnp.tanh(agg), wn_ref[...], preferred_element_type=jnp.float32),
        0.0,
    )
    b = b_ref[0, 0, :]
    onehot = (lax.broadcasted_iota(jnp.int32, (_G, _RB), 0) == b[None, :])
    onehot = onehot.astype(jnp.float32)
    pooled[...] += jnp.dot(onehot, o, preferred_element_type=jnp.float32)
    counts[...] += jnp.sum(onehot, axis=1, keepdims=True)

    @pl.when(i == _GRID - 1)
    def _():
        pm = pooled[...] / jnp.maximum(counts[...], 1.0)
        logits = jnp.dot(pm, wp_ref[...], preferred_element_type=jnp.float32)
        logits = logits + bp_ref[...]
        m = jnp.max(logits, axis=-1, keepdims=True)
        lse = jnp.log(jnp.sum(jnp.exp(logits - m), axis=-1, keepdims=True)) + m
        out_ref[...] = logits - lse


def _tc2(xw, agg2, w_nbr_t, batch_r, w_pred_t, b_pred_r):
    return pl.pallas_call(
        _tc2_body,
        grid=(_GRID,),
        in_specs=[
            pl.BlockSpec((_RB, _HID), lambda i: (i, 0)),
            pl.BlockSpec((_NC, _RB, _HID), lambda i: (0, i, 0)),
            pl.BlockSpec((_HID, _HID), lambda i: (0, 0)),
            pl.BlockSpec((1, 1, _RB), lambda i: (i, 0, 0)),
            pl.BlockSpec((_HID, _OUT), lambda i: (0, 0)),
            pl.BlockSpec((1, _OUT), lambda i: (0, 0)),
        ],
        out_specs=pl.BlockSpec((_G, _OUT), lambda i: (0, 0)),
        out_shape=jax.ShapeDtypeStruct((_G, _OUT), jnp.float32),
        scratch_shapes=[
            pltpu.VMEM((_G, _HID), jnp.float32),
            pltpu.VMEM((_G, 1), jnp.float32),
        ],
    )(xw, agg2, w_nbr_t, batch_r, w_pred_t, b_pred_r)


def kernel(x, edge_index, batch, W_self, W_nbr, W_pred, b_pred):
    xw, t = _tc1(x, W_self.T)
    e6 = edge_index.reshape(2, _NC, _NS, _NGRP, _GP, _K)
    agg2 = _sc_scatter(e6, t)
    batch_r = batch.reshape(_GRID, 1, _RB)
    return _tc2(xw, agg2, W_nbr.T, batch_r, W_pred.T, b_pred.reshape(1, _OUT))


# speedup vs baseline: 1.0607x; 1.0023x over previous
import jax
import jax.numpy as jnp
from jax import lax
from jax.experimental import pallas as pl
from jax.experimental.pallas import tpu as pltpu
from jax.experimental.pallas import tpu_sc as plsc

_N = 10000
_E = 320000
_HID = 128
_OUT = 10
_G = 64

_RB = 1000
_GRID = _N // _RB

_NC, _NS = 2, 16
_K = 125
_EPT = _E // (_NC * _NS)
_CHUNKS = _EPT // _K
_GP = 8
_NGRP = _CHUNKS // _GP
_NPAD = 10240
_RPT = _NPAD // _NS
_RCH = 80


def _tc1_body(x_ref, w_ref, xw_ref, t_ref):
    xw = jnp.dot(x_ref[...], w_ref[...], preferred_element_type=jnp.float32)
    xw_ref[...] = xw
    t_ref[...] = jnp.tanh(jnp.maximum(xw, 0.0))


def _tc1(x, w_self_t):
    return pl.pallas_call(
        _tc1_body,
        grid=(_GRID,),
        in_specs=[
            pl.BlockSpec((_RB, _HID), lambda i: (i, 0)),
            pl.BlockSpec((_HID, _HID), lambda i: (0, 0)),
        ],
        out_specs=[
            pl.BlockSpec((_RB, _HID), lambda i: (i, 0)),
            pl.BlockSpec((_RB, _HID), lambda i: (i, 0)),
        ],
        out_shape=[
            jax.ShapeDtypeStruct((_N, _HID), jnp.float32),
            jax.ShapeDtypeStruct((_N, _HID), jnp.float32),
        ],
    )(x, w_self_t)


def _sc_body(e_ref, t_ref, out_ref, sia, dia, sib, dib, gba, gbb,
             acc, semga, semgb, semsa, semsb, semi):
    cid = lax.axis_index("c")
    sid = lax.axis_index("s")
    src_ref = e_ref.at[0]
    dst_ref = e_ref.at[1]

    def _zrow(r, c0):
        def _zcol(c, c1):
            gba[r, pl.ds(c * 16, 16)] = jnp.zeros((16,), jnp.float32)
            return c1
        return lax.fori_loop(0, _HID // 16, _zcol, c0)

    lax.fori_loop(0, _RCH, _zrow, 0)

    def _zslice(q, c0):
        pltpu.sync_copy(gba.at[pl.ds(0, _RCH)],
                        acc.at[pl.ds(sid * _RPT + q * _RCH, _RCH)])
        return c0

    lax.fori_loop(0, _RPT // _RCH, _zslice, 0)
    plsc.subcore_barrier()

    pltpu.sync_copy(src_ref.at[cid, sid, 0], sia)
    pltpu.sync_copy(dst_ref.at[cid, sid, 0], dia)
    pltpu.async_copy(t_ref.at[sia.at[0]], gba, semga)

    def _do_group(cs, cd, crossing, skip_pred):
        for r in range(_GP):
            if r % 2 == 0:
                buf, semg, sems = gba, semga, semsa
                obuf, osemg, osems = gbb, semgb, semsb
            else:
                buf, semg, sems = gbb, semgb, semsb
                obuf, osemg, osems = gba, semga, semsa
            pltpu.make_async_copy(t_ref.at[cs.at[r]], buf, semg).wait()
            pltpu.async_copy(buf, acc.at[cd.at[r]], sems, add=True)

            def _wait_prev(ob=obuf, os=osems, rr=r):
                pltpu.make_async_copy(ob, acc.at[cd.at[rr]], os).wait()

            if r == 0 and skip_pred is not None:
                @pl.when(jnp.logical_not(skip_pred))
                def _():
                    _wait_prev()
            else:
                _wait_prev()
            if r < _GP - 1:
                pltpu.async_copy(t_ref.at[cs.at[r + 1]], obuf, osemg)
            elif crossing is not None:
                crossing()

    def _cross(ns, nd):
        pltpu.make_async_copy(src_ref.at[cid, sid, 0], ns, semi).wait()
        pltpu.make_async_copy(dst_ref.at[cid, sid, 0], nd, semi).wait()
        pltpu.async_copy(t_ref.at[ns.at[0]], gba, semga)

    def _pairbody(gg, c0):
        g0 = 2 * gg
        pltpu.async_copy(src_ref.at[cid, sid, g0 + 1], sib, semi)
        pltpu.async_copy(dst_ref.at[cid, sid, g0 + 1], dib, semi)
        _do_group(sia, dia, lambda: _cross(sib, dib), gg == 0)

        @pl.when(gg < _NGRP // 2 - 1)
        def _():
            pltpu.async_copy(src_ref.at[cid, sid, g0 + 2], sia, semi)
            pltpu.async_copy(dst_ref.at[cid, sid, g0 + 2], dia, semi)

        def _cross_if_more():
            @pl.when(gg < _NGRP // 2 - 1)
            def _():
                _cross(sia, dia)

        _do_group(sib, dib, _cross_if_more, None)
        return c0

    lax.fori_loop(0, _NGRP // 2, _pairbody, 0)
    pltpu.make_async_copy(gbb, acc.at[dib.at[_GP - 1]], semsb).wait()
    plsc.subcore_barrier()

    r0 = sid * _RPT
    pltpu.sync_copy(acc.at[pl.ds(r0, _RPT)], out_ref.at[cid, pl.ds(r0, _RPT)])


def _sc_scatter(e6, t):
    mesh = plsc.VectorSubcoreMesh(core_axis_name="c", subcore_axis_name="s")
    f = pl.kernel(
        _sc_body,
        mesh=mesh,
        out_type=jax.ShapeDtypeStruct((_NC, _NPAD, _HID), jnp.float32),
        scratch_types=[
            pltpu.VMEM((_GP, _K), jnp.int32),
            pltpu.VMEM((_GP, _K), jnp.int32),
            pltpu.VMEM((_GP, _K), jnp.int32),
            pltpu.VMEM((_GP, _K), jnp.int32),
            pltpu.VMEM((_K, _HID), jnp.float32),
            pltpu.VMEM((_K, _HID), jnp.float32),
            pltpu.VMEM_SHARED((_NPAD, _HID), jnp.float32),
            pltpu.SemaphoreType.DMA,
            pltpu.SemaphoreType.DMA,
            pltpu.SemaphoreType.DMA,
            pltpu.SemaphoreType.DMA,
            pltpu.SemaphoreType.DMA,
        ],
    )
    return f(e6, t)


def _tc2_body(xw_ref, agg_ref, wn_ref, b_ref, wp_ref, bp_ref, out_ref,
              pooled, counts):
    i = pl.program_id(0)

    @pl.when(i == 0)
    def _():
        pooled[...] = jnp.zeros_like(pooled)
        counts[...] = jnp.zeros_like(counts)

    agg = agg_ref[0] + agg_ref[1]
    o = jnp.maximum(
        xw_ref[...]
        + jnp.dot(jnp.tanh(agg), wn_ref[...], preferred_element_type=jnp.float32),
        0.0,
    )
    b = b_ref[0, 0, :]
    onehot = (lax.broadcasted_iota(jnp.int32, (_G, _RB), 0) == b[None, :])
    onehot = onehot.astype(jnp.float32)
    pooled[...] += jnp.dot(onehot, o, preferred_element_type=jnp.float32)
    counts[...] += jnp.sum(onehot, axis=1, keepdims=True)

    @pl.when(i == _GRID - 1)
    def _():
        pm = pooled[...] / jnp.maximum(counts[...], 1.0)
        logits = jnp.dot(pm, wp_ref[...], preferred_element_type=jnp.float32)
        logits = logits + bp_ref[...]
        m = jnp.max(logits, axis=-1, keepdims=True)
        lse = jnp.log(jnp.sum(jnp.exp(logits - m), axis=-1, keepdims=True)) + m
        out_ref[...] = logits - lse


def _tc2(xw, agg2, w_nbr_t, batch_r, w_pred_t, b_pred_r):
    return pl.pallas_call(
        _tc2_body,
        grid=(_GRID,),
        in_specs=[
            pl.BlockSpec((_RB, _HID), lambda i: (i, 0)),
            pl.BlockSpec((_NC, _RB, _HID), lambda i: (0, i, 0)),
            pl.BlockSpec((_HID, _HID), lambda i: (0, 0)),
            pl.BlockSpec((1, 1, _RB), lambda i: (i, 0, 0)),
            pl.BlockSpec((_HID, _OUT), lambda i: (0, 0)),
            pl.BlockSpec((1, _OUT), lambda i: (0, 0)),
        ],
        out_specs=pl.BlockSpec((_G, _OUT), lambda i: (0, 0)),
        out_shape=jax.ShapeDtypeStruct((_G, _OUT), jnp.float32),
        scratch_shapes=[
            pltpu.VMEM((_G, _HID), jnp.float32),
            pltpu.VMEM((_G, 1), jnp.float32),
        ],
    )(xw, agg2, w_nbr_t, batch_r, w_pred_t, b_pred_r)


def kernel(x, edge_index, batch, W_self, W_nbr, W_pred, b_pred):
    xw, t = _tc1(x, W_self.T)
    e6 = edge_index.reshape(2, _NC, _NS, _NGRP, _GP, _K)
    agg2 = _sc_scatter(e6, t)
    batch_r = batch.reshape(_GRID, 1, _RB)
    return _tc2(xw, agg2, W_nbr.T, batch_r, W_pred.T, b_pred.reshape(1, _OUT))
